# baseline (speedup 1.0000x reference)
import jax
import jax.numpy as jnp
from jax.experimental import pallas as pl
from jax.experimental.pallas import tpu as pltpu

_ROWS = 32768
_COLS = 1024
_BLK = 512
_NSPLIT = 4
_PART = _ROWS // _NSPLIT
_STEPS = _PART // _BLK


def _tc_body(*refs):
    x_refs = refs[:_NSPLIT]
    y_refs = refs[_NSPLIT : 2 * _NSPLIT]
    nx_ref, ny_ref = refs[2 * _NSPLIT], refs[2 * _NSPLIT + 1]
    accx, accy = refs[2 * _NSPLIT + 2], refs[2 * _NSPLIT + 3]
    i = pl.program_id(0)

    @pl.when(i == 0)
    def _init():
        accx[...] = jnp.zeros_like(accx)
        accy[...] = jnp.zeros_like(accy)

    def csum(ref):
        s = (ref[...] > 0).astype(jnp.int32).reshape(_BLK // 8, 8, _COLS)
        return jnp.sum(s, axis=0)

    ax = csum(x_refs[0])
    ay = csum(y_refs[0])
    for k in range(1, _NSPLIT):
        ax = ax + csum(x_refs[k])
        ay = ay + csum(y_refs[k])
    accx[...] += ax
    accy[...] += ay

    @pl.when(i == _STEPS - 1)
    def _fin():
        nx_ref[0, 0] = jnp.sum(accx[...])
        ny_ref[0, 0] = jnp.sum(accy[...])


def kernel(x, y):
    def part(k):
        return pl.BlockSpec((_BLK, _COLS), lambda i, k=k: (i + k * _STEPS, 0))

    specs = [part(k) for k in range(_NSPLIT)]
    nx, ny = pl.pallas_call(
        _tc_body,
        grid=(_STEPS,),
        in_specs=specs + specs,
        out_specs=[
            pl.BlockSpec(memory_space=pltpu.SMEM),
            pl.BlockSpec(memory_space=pltpu.SMEM),
        ],
        out_shape=[
            jax.ShapeDtypeStruct((1, 1), jnp.int32),
            jax.ShapeDtypeStruct((1, 1), jnp.int32),
        ],
        scratch_shapes=[
            pltpu.VMEM((8, _COLS), jnp.int32),
            pltpu.VMEM((8, _COLS), jnp.int32),
        ],
    )(*([x] * _NSPLIT + [y] * _NSPLIT))
    return jnp.maximum(nx[0, 0], ny[0, 0])


# speedup vs baseline: 1.0251x; 1.0251x over previous
import jax
import jax.numpy as jnp
from jax.experimental import pallas as pl
from jax.experimental.pallas import tpu as pltpu

_ROWS = 32768
_COLS = 1024
_BLK = 256
_NSPLIT = 2
_PART = _ROWS // _NSPLIT
_STEPS = _PART // _BLK


def _tc_body(*refs):
    x_refs = refs[:_NSPLIT]
    y_refs = refs[_NSPLIT : 2 * _NSPLIT]
    nx_ref, ny_ref = refs[2 * _NSPLIT], refs[2 * _NSPLIT + 1]
    accx, accy = refs[2 * _NSPLIT + 2], refs[2 * _NSPLIT + 3]
    i = pl.program_id(0)

    @pl.when(i == 0)
    def _init():
        accx[...] = jnp.zeros_like(accx)
        accy[...] = jnp.zeros_like(accy)

    def csum(ref):
        s = (ref[...] > 0).astype(jnp.int32).reshape(_BLK // 8, 8, _COLS)
        return jnp.sum(s, axis=0)

    ax = csum(x_refs[0])
    ay = csum(y_refs[0])
    for k in range(1, _NSPLIT):
        ax = ax + csum(x_refs[k])
        ay = ay + csum(y_refs[k])
    accx[...] += ax
    accy[...] += ay

    @pl.when(i == _STEPS - 1)
    def _fin():
        nx_ref[0, 0] = jnp.sum(accx[...])
        ny_ref[0, 0] = jnp.sum(accy[...])


def kernel(x, y):
    def part(k):
        return pl.BlockSpec((_BLK, _COLS), lambda i, k=k: (i + k * _STEPS, 0))

    specs = [part(k) for k in range(_NSPLIT)]
    nx, ny = pl.pallas_call(
        _tc_body,
        grid=(_STEPS,),
        in_specs=specs + specs,
        out_specs=[
            pl.BlockSpec(memory_space=pltpu.SMEM),
            pl.BlockSpec(memory_space=pltpu.SMEM),
        ],
        out_shape=[
            jax.ShapeDtypeStruct((1, 1), jnp.int32),
            jax.ShapeDtypeStruct((1, 1), jnp.int32),
        ],
        scratch_shapes=[
            pltpu.VMEM((8, _COLS), jnp.int32),
            pltpu.VMEM((8, _COLS), jnp.int32),
        ],
    )(*([x] * _NSPLIT + [y] * _NSPLIT))
    return jnp.maximum(nx[0, 0], ny[0, 0])


# speedup vs baseline: 1.1244x; 1.0968x over previous
import jax
import jax.numpy as jnp
from jax.experimental import pallas as pl
from jax.experimental.pallas import tpu as pltpu

_ROWS = 32768
_COLS = 1024
_BLK = 1024
_NSPLIT = 2
_PART = _ROWS // _NSPLIT
_STEPS = _PART // _BLK


def _tc_body(*refs):
    x_refs = refs[:_NSPLIT]
    y_refs = refs[_NSPLIT : 2 * _NSPLIT]
    nx_ref, ny_ref = refs[2 * _NSPLIT], refs[2 * _NSPLIT + 1]
    accx, accy = refs[2 * _NSPLIT + 2], refs[2 * _NSPLIT + 3]
    i = pl.program_id(0)

    @pl.when(i == 0)
    def _init():
        accx[...] = jnp.zeros_like(accx)
        accy[...] = jnp.zeros_like(accy)

    def csum(ref):
        s = (ref[...] > 0).astype(jnp.int32).reshape(_BLK // 8, 8, _COLS)
        return jnp.sum(s, axis=0)

    ax = csum(x_refs[0])
    ay = csum(y_refs[0])
    for k in range(1, _NSPLIT):
        ax = ax + csum(x_refs[k])
        ay = ay + csum(y_refs[k])
    accx[...] += ax
    accy[...] += ay

    @pl.when(i == _STEPS - 1)
    def _fin():
        nx_ref[0, 0] = jnp.sum(accx[...])
        ny_ref[0, 0] = jnp.sum(accy[...])


def kernel(x, y):
    def part(k):
        return pl.BlockSpec((_BLK, _COLS), lambda i, k=k: (i + k * _STEPS, 0))

    specs = [part(k) for k in range(_NSPLIT)]
    nx, ny = pl.pallas_call(
        _tc_body,
        grid=(_STEPS,),
        in_specs=specs + specs,
        out_specs=[
            pl.BlockSpec(memory_space=pltpu.SMEM),
            pl.BlockSpec(memory_space=pltpu.SMEM),
        ],
        out_shape=[
            jax.ShapeDtypeStruct((1, 1), jnp.int32),
            jax.ShapeDtypeStruct((1, 1), jnp.int32),
        ],
        scratch_shapes=[
            pltpu.VMEM((8, _COLS), jnp.int32),
            pltpu.VMEM((8, _COLS), jnp.int32),
        ],
    )(*([x] * _NSPLIT + [y] * _NSPLIT))
    return jnp.maximum(nx[0, 0], ny[0, 0])
